# Initial kernel scaffold; baseline (speedup 1.0000x reference)
import jax
import jax.numpy as jnp
from jax.experimental import pallas as pl


def kernel(phi):
    raise NotImplementedError("write your pallas kernel here")


# speedup vs baseline: 12.1342x; 12.1342x over previous
import jax
import jax.numpy as jnp
from jax import lax
from jax.experimental import pallas as pl

_BB = 8


def _body(x_ref, o_ref):
    x = x_ref[...]
    r = lax.broadcasted_iota(jnp.int32, x.shape, 1)
    c = lax.broadcasted_iota(jnp.int32, x.shape, 2)
    mask = ((r + c) % 2) == 0
    a = jnp.where(mask, x, 0.0)
    b = jnp.where(mask, 0.0, x)
    o_ref[...] = a + b


def kernel(phi):
    B, H, W = phi.shape
    return pl.pallas_call(
        _body,
        grid=(B // _BB,),
        in_specs=[pl.BlockSpec((_BB, H, W), lambda i: (i, 0, 0))],
        out_specs=pl.BlockSpec((_BB, H, W), lambda i: (i, 0, 0)),
        out_shape=jax.ShapeDtypeStruct(phi.shape, phi.dtype),
    )(phi)
